# baseline (speedup 1.0000x reference)
import jax
import jax.numpy as jnp
from jax import lax
from jax.experimental import pallas as pl
from jax.experimental.pallas import tpu as pltpu
from jax.experimental.pallas import tpu_sc as plsc

D = 1024
S = 2048
B = 4
NC, NS, L = 2, 16, 16
NW = NC * NS
T = B * S
TPW = T // NW
C = 32
NCHUNK = TPW // C
WPS = S // TPW


def _emb_body(ids_hbm, table_hbm, pos_hbm, out_hbm, idx_v, rows_v, pos_v, gsem):
    wid = lax.axis_index("s") * NC + lax.axis_index("c")
    base = wid * TPW
    pos_base = lax.rem(wid, WPS) * TPW

    pltpu.sync_copy(ids_hbm.at[wid], idx_v)

    for c in range(NCHUNK):
        gather = pltpu.async_copy(table_hbm.at[idx_v.at[c]], rows_v, gsem)
        pltpu.sync_copy(pos_hbm.at[pl.ds(pos_base + c * C, C)], pos_v)
        gather.wait()

        def add_row(r, carry):
            for j in range(D // L):
                plsc.addupdate(rows_v.at[r, pl.ds(j * L, L)],
                               pos_v[r, pl.ds(j * L, L)])
            return carry

        lax.fori_loop(0, C, add_row, 0)
        pltpu.sync_copy(rows_v, out_hbm.at[pl.ds(base + c * C, C)])


def kernel(input_ids, word_embeddings, position_embeddings):
    ids = input_ids.astype(jnp.int32).reshape(NW, NCHUNK, C)
    mesh = plsc.VectorSubcoreMesh(core_axis_name="c", subcore_axis_name="s",
                                  num_cores=NC, num_subcores=NS)
    out = pl.kernel(
        _emb_body,
        out_type=jax.ShapeDtypeStruct((T, D), jnp.float32),
        mesh=mesh,
        scratch_types=[
            pltpu.VMEM((NCHUNK, C), jnp.int32),
            pltpu.VMEM((C, D), jnp.float32),
            pltpu.VMEM((C, D), jnp.float32),
            pltpu.SemaphoreType.DMA,
        ],
    )(ids, word_embeddings, position_embeddings)
    return out.reshape(B, S, D)


# speedup vs baseline: 1.3731x; 1.3731x over previous
import jax
import jax.numpy as jnp
from jax import lax
from jax.experimental import pallas as pl
from jax.experimental.pallas import tpu as pltpu
from jax.experimental.pallas import tpu_sc as plsc

D = 1024
S = 2048
B = 4
NC, NS, L = 2, 16, 16
NW = NC * NS
POSW = S // NW
CR = 16
CPB = POSW // CR
NCHUNK = B * CPB
NBUF = 3


def _emb_body(ids_hbm, table_hbm, pos_hbm, out_hbm, idx_v, pos_v, rows_v,
              gsem, psem, osem):
    wid = lax.axis_index("s") * NC + lax.axis_index("c")

    pltpu.sync_copy(ids_hbm.at[wid], idx_v)

    gd = [None] * NCHUNK
    sd = [None] * NCHUNK
    for cc in range(min(NBUF - 1, NCHUNK)):
        gd[cc] = pltpu.async_copy(table_hbm.at[idx_v.at[cc]],
                                  rows_v.at[cc % NBUF], gsem)

    pos_cp = pltpu.async_copy(pos_hbm.at[pl.ds(wid * POSW, POSW)], pos_v, psem)

    for cc in range(NCHUNK):
        b, c = divmod(cc, CPB)
        buf = cc % NBUF
        nxt = cc + NBUF - 1
        if nxt < NCHUNK:
            if cc >= 1:
                sd[cc - 1].wait()
            gd[nxt] = pltpu.async_copy(table_hbm.at[idx_v.at[nxt]],
                                       rows_v.at[nxt % NBUF], gsem)
        if cc == 0:
            pos_cp.wait()
        gd[cc].wait()

        def add_row(r, carry):
            for j in range(D // L):
                plsc.addupdate(rows_v.at[buf, r, pl.ds(j * L, L)],
                               pos_v[c * CR + r, pl.ds(j * L, L)])
            return carry

        lax.fori_loop(0, CR, add_row, 0)
        base = b * S + wid * POSW + c * CR
        sd[cc] = pltpu.async_copy(rows_v.at[buf],
                                  out_hbm.at[pl.ds(base, CR)], osem)

    for cc in range(max(0, NCHUNK - NBUF), NCHUNK):
        sd[cc].wait()


def kernel(input_ids, word_embeddings, position_embeddings):
    ids = (input_ids.astype(jnp.int32)
           .reshape(B, NW, CPB, CR)
           .transpose(1, 0, 2, 3)
           .reshape(NW, NCHUNK, CR))
    mesh = plsc.VectorSubcoreMesh(core_axis_name="c", subcore_axis_name="s",
                                  num_cores=NC, num_subcores=NS)
    out = pl.kernel(
        _emb_body,
        out_type=jax.ShapeDtypeStruct((B * S, D), jnp.float32),
        mesh=mesh,
        scratch_types=[
            pltpu.VMEM((NCHUNK, CR), jnp.int32),
            pltpu.VMEM((POSW, D), jnp.float32),
            pltpu.VMEM((NBUF, CR, D), jnp.float32),
            pltpu.SemaphoreType.DMA,
            pltpu.SemaphoreType.DMA,
            pltpu.SemaphoreType.DMA,
        ],
    )(ids, word_embeddings, position_embeddings)
    return out.reshape(B, S, D)


# speedup vs baseline: 1.7520x; 1.2760x over previous
import jax
import jax.numpy as jnp
from jax import lax
from jax.experimental import pallas as pl
from jax.experimental.pallas import tpu as pltpu
from jax.experimental.pallas import tpu_sc as plsc

D = 1024
S = 2048
B = 4
NC, NS, L = 2, 16, 16
NW = NC * NS
POSW = S // NW
CR = 32
CPB = POSW // CR
NCHUNK = CPB * B
NBUF = 2


def _emb_body(ids_hbm, table_hbm, pos_hbm, out_hbm, idx_v, pos_v, rows_v,
              gsem, psem, osem):
    wid = lax.axis_index("s") * NC + lax.axis_index("c")

    pltpu.sync_copy(ids_hbm.at[wid], idx_v)

    gd = [None] * NCHUNK
    sd = [None] * NCHUNK
    gd[0] = pltpu.async_copy(table_hbm.at[idx_v.at[0]], rows_v.at[0], gsem)
    pos_cp = pltpu.async_copy(pos_hbm.at[pl.ds(wid * POSW, CR)], pos_v, psem)

    for cc in range(NCHUNK):
        c, b = divmod(cc, B)
        buf = cc % NBUF
        if cc + 1 < NCHUNK:
            if cc >= 1:
                sd[cc - 1].wait()
            gd[cc + 1] = pltpu.async_copy(table_hbm.at[idx_v.at[cc + 1]],
                                          rows_v.at[(cc + 1) % NBUF], gsem)
        if cc % B == 0:
            pos_cp.wait()
        gd[cc].wait()

        def add_row(r):
            for j in range(D // L):
                plsc.addupdate(rows_v.at[buf, r, pl.ds(j * L, L)],
                               pos_v[r, pl.ds(j * L, L)])

        plsc.parallel_loop(0, CR, unroll=2)(add_row)

        if b == B - 1 and c + 1 < CPB:
            pos_cp = pltpu.async_copy(
                pos_hbm.at[pl.ds(wid * POSW + (c + 1) * CR, CR)], pos_v, psem)

        base = b * S + wid * POSW + c * CR
        sd[cc] = pltpu.async_copy(rows_v.at[buf],
                                  out_hbm.at[pl.ds(base, CR)], osem)

    for cc in range(max(0, NCHUNK - NBUF), NCHUNK):
        sd[cc].wait()


def kernel(input_ids, word_embeddings, position_embeddings):
    ids = (input_ids.astype(jnp.int32)
           .reshape(B, NW, CPB, CR)
           .transpose(1, 2, 0, 3)
           .reshape(NW, NCHUNK, CR))
    mesh = plsc.VectorSubcoreMesh(core_axis_name="c", subcore_axis_name="s",
                                  num_cores=NC, num_subcores=NS)
    out = pl.kernel(
        _emb_body,
        out_type=jax.ShapeDtypeStruct((B * S, D), jnp.float32),
        mesh=mesh,
        scratch_types=[
            pltpu.VMEM((NCHUNK, CR), jnp.int32),
            pltpu.VMEM((CR, D), jnp.float32),
            pltpu.VMEM((NBUF, CR, D), jnp.float32),
            pltpu.SemaphoreType.DMA,
            pltpu.SemaphoreType.DMA,
            pltpu.SemaphoreType.DMA,
        ],
    )(ids, word_embeddings, position_embeddings)
    return out.reshape(B, S, D)


# speedup vs baseline: 2.5137x; 1.4348x over previous
import jax
import jax.numpy as jnp
from jax import lax
from jax.experimental import pallas as pl
from jax.experimental.pallas import tpu as pltpu
from jax.experimental.pallas import tpu_sc as plsc

D = 1024
S = 2048
B = 4
NC, NS, L = 2, 16, 16
NW = NC * NS
POSW = S // NW
CR = 16
CPB = POSW // CR
NCHUNK = CPB * B
NBUF = 5
KAHEAD = 3


def _emb_body(ids_hbm, table_hbm, pos_hbm, out_hbm, idx_v, pos_v, rows_v,
              *sems):
    gsem = sems[:NBUF]
    osem = sems[NBUF:2 * NBUF]
    psem = sems[2 * NBUF]

    wid = lax.axis_index("s") * NC + lax.axis_index("c")

    pltpu.sync_copy(ids_hbm.at[wid], idx_v)

    gd = [None] * NCHUNK
    sd = [None] * NCHUNK
    pd = [None] * CPB
    for cc in range(min(KAHEAD, NCHUNK)):
        gd[cc] = pltpu.async_copy(table_hbm.at[idx_v.at[cc]],
                                  rows_v.at[cc % NBUF], gsem[cc % NBUF])
    pd[0] = pltpu.async_copy(pos_hbm.at[pl.ds(wid * POSW, CR)],
                             pos_v.at[0], psem)

    for cc in range(NCHUNK):
        c, b = divmod(cc, B)
        buf = cc % NBUF
        nxt = cc + KAHEAD
        if nxt < NCHUNK:
            if nxt >= NBUF:
                sd[nxt - NBUF].wait()
            gd[nxt] = pltpu.async_copy(table_hbm.at[idx_v.at[nxt]],
                                       rows_v.at[nxt % NBUF], gsem[nxt % NBUF])
        if b == 0:
            pd[c].wait()
            if c + 1 < CPB:
                pd[c + 1] = pltpu.async_copy(
                    pos_hbm.at[pl.ds(wid * POSW + (c + 1) * CR, CR)],
                    pos_v.at[(c + 1) % 2], psem)
        gd[cc].wait()

        JPR = D // L

        def add_grp(i):
            r = lax.shift_right_logical(i, 6)
            off = (i & (JPR - 1)) * L
            plsc.addupdate(rows_v.at[buf, r, pl.ds(off, L)],
                           pos_v[c % 2, r, pl.ds(off, L)])

        plsc.parallel_loop(0, CR * JPR, unroll=8)(add_grp)

        base = b * S + wid * POSW + c * CR
        sd[cc] = pltpu.async_copy(rows_v.at[buf],
                                  out_hbm.at[pl.ds(base, CR)], osem[buf])

    for cc in range(max(0, NCHUNK - NBUF), NCHUNK):
        sd[cc].wait()


def kernel(input_ids, word_embeddings, position_embeddings):
    ids = (input_ids.astype(jnp.int32)
           .reshape(B, NW, CPB, CR)
           .transpose(1, 2, 0, 3)
           .reshape(NW, NCHUNK, CR))
    mesh = plsc.VectorSubcoreMesh(core_axis_name="c", subcore_axis_name="s",
                                  num_cores=NC, num_subcores=NS)
    out = pl.kernel(
        _emb_body,
        out_type=jax.ShapeDtypeStruct((B * S, D), jnp.float32),
        mesh=mesh,
        scratch_types=[
            pltpu.VMEM((NCHUNK, CR), jnp.int32),
            pltpu.VMEM((2, CR, D), jnp.float32),
            pltpu.VMEM((NBUF, CR, D), jnp.float32),
        ] + [pltpu.SemaphoreType.DMA] * (2 * NBUF + 1),
    )(ids, word_embeddings, position_embeddings)
    return out.reshape(B, S, D)
